# initial kernel scaffold (unmeasured)
import jax
import jax.numpy as jnp
from jax import lax
from jax.experimental import pallas as pl
from jax.experimental.pallas import tpu as pltpu

N_DEV = 4
M, K, N = 4096, 1024, 8192
MC = M // N_DEV
NH = N // 2
OT = 1024


def _gemm(x, w):
    BM, BN = 1024, 2048

    def body(x_ref, w_ref, o_ref):
        a = x_ref[:, :].astype(jnp.bfloat16)
        b = w_ref[:, :].astype(jnp.bfloat16)
        o_ref[:, :] = jnp.dot(
            a, b, preferred_element_type=jnp.float32
        ).astype(jnp.bfloat16)

    return pl.pallas_call(
        body,
        grid=(M // BM, N // BN),
        in_specs=[
            pl.BlockSpec((BM, K), lambda i, j: (i, 0)),
            pl.BlockSpec((K, BN), lambda i, j: (0, j)),
        ],
        out_specs=pl.BlockSpec((BM, BN), lambda i, j: (i, j)),
        out_shape=jax.ShapeDtypeStruct((M, N), jnp.bfloat16),
    )(x, w)


def _ar_body(p_ref, scale_ref, out_ref, recv, pstage, ostage,
             send_sems, recv_sems, load_sems, ostage_sems, credit_sems):
    d = lax.axis_index("i")
    scale = scale_ref[0, 0]
    DIRS = (1, -1)
    halfbase = (0, NH)
    nbr = [jnp.mod(d + 1, N_DEV), jnp.mod(d - 1, N_DEV)]
    peer_out = [nbr[0], nbr[1]]
    peer_in = [nbr[1], nbr[0]]

    def rows(c):
        return pl.ds(c * MC, MC)

    preloads = []
    pfetch = [None, None]
    for k, dirn in enumerate(DIRS):
        cp = pltpu.make_async_copy(
            p_ref.at[rows(d), pl.ds(halfbase[k], NH)],
            recv.at[k, 0], load_sems.at[k, 0])
        cp.start()
        preloads.append(cp)
        cr0 = jnp.mod(d - dirn, N_DEV)
        cp = pltpu.make_async_copy(
            p_ref.at[rows(cr0), pl.ds(halfbase[k], NH)],
            pstage.at[k], load_sems.at[k, 1])
        cp.start()
        pfetch[k] = cp

    barrier = pltpu.get_barrier_semaphore()
    for k in range(2):
        pl.semaphore_signal(barrier, inc=1, device_id=(nbr[k],),
                            device_id_type=pl.DeviceIdType.MESH)
    pl.semaphore_wait(barrier, 2)
    for cp in preloads:
        cp.wait()

    last_ocp = [None, None]

    def epilogue(k, slot, c):
        for j in range(NH // OT):
            if last_ocp[k] is not None:
                last_ocp[k].wait()
            v = recv[k, slot, :, j * OT:(j + 1) * OT].astype(jnp.float32)
            ostage[k, :, :] = jnp.maximum(v * scale, 0.0)
            cp = pltpu.make_async_copy(
                ostage.at[k],
                out_ref.at[rows(c), pl.ds(halfbase[k] + j * OT, OT)],
                ostage_sems.at[k])
            cp.start()
            last_ocp[k] = cp

    rdmas = [[None] * 6 for _ in range(2)]
    for t in range(6):
        S, D = t % 2, (t + 1) % 2
        for k, dirn in enumerate(DIRS):
            if t >= 1:
                pl.semaphore_wait(credit_sems.at[k], 1)
            r = pltpu.make_async_remote_copy(
                src_ref=recv.at[k, S],
                dst_ref=recv.at[k, D],
                send_sem=send_sems.at[k, S],
                recv_sem=recv_sems.at[k, D],
                device_id=(peer_out[k],),
                device_id_type=pl.DeviceIdType.MESH)
            r.start()
            rdmas[k][t] = r
        if t - 1 >= 2:
            for k, dirn in enumerate(DIRS):
                tp = t - 1
                c = (jnp.mod(d + dirn, N_DEV) if tp == 2
                     else jnp.mod(d - dirn * (tp - 3), N_DEV))
                epilogue(k, (tp + 1) % 2, c)
        for k, dirn in enumerate(DIRS):
            if t <= 2:
                pfetch[k].wait()
            rdmas[k][t].wait_recv()
            if t <= 2:
                acc = (recv[k, D, :, :].astype(jnp.float32)
                       + pstage[k, :, :].astype(jnp.float32))
                recv[k, D, :, :] = acc.astype(jnp.bfloat16)
                if t < 2:
                    crn = jnp.mod(d - dirn * (t + 2), N_DEV)
                    cp = pltpu.make_async_copy(
                        p_ref.at[rows(crn), pl.ds(halfbase[k], NH)],
                        pstage.at[k], load_sems.at[k, 1])
                    cp.start()
                    pfetch[k] = cp
            rdmas[k][t].wait_send()
            pl.semaphore_signal(credit_sems.at[k], inc=1,
                                device_id=(peer_in[k],),
                                device_id_type=pl.DeviceIdType.MESH)
    for k, dirn in enumerate(DIRS):
        epilogue(k, 0, jnp.mod(d - dirn * 2, N_DEV))
    for k in range(2):
        if last_ocp[k] is not None:
            last_ocp[k].wait()
        pl.semaphore_wait(credit_sems.at[k], 1)


def _all_reduce_relu(partial, scale):
    return pl.pallas_call(
        _ar_body,
        in_specs=[
            pl.BlockSpec(memory_space=pltpu.ANY),
            pl.BlockSpec(memory_space=pltpu.SMEM),
        ],
        out_specs=pl.BlockSpec(memory_space=pltpu.ANY),
        out_shape=jax.ShapeDtypeStruct((M, N), jnp.float32),
        scratch_shapes=[
            pltpu.VMEM((2, 2, MC, NH), jnp.bfloat16),
            pltpu.VMEM((2, MC, NH), jnp.bfloat16),
            pltpu.VMEM((2, MC, OT), jnp.float32),
            pltpu.SemaphoreType.DMA((2, 2)),
            pltpu.SemaphoreType.DMA((2, 2)),
            pltpu.SemaphoreType.DMA((2, 2)),
            pltpu.SemaphoreType.DMA((2,)),
            pltpu.SemaphoreType.REGULAR((2,)),
        ],
        compiler_params=pltpu.CompilerParams(collective_id=0),
    )(partial, scale)


def kernel(x, w_mat, scale_x, scale_w):
    partial = _gemm(x, w_mat)
    scale = (scale_x * scale_w).reshape(1, 1).astype(jnp.float32)
    return _all_reduce_relu(partial, scale)


# baseline (device time: 765401 ns/iter reference)
import jax
import jax.numpy as jnp
from jax import lax
from jax.experimental import pallas as pl
from jax.experimental.pallas import tpu as pltpu

N_DEV = 4
M, K, N = 4096, 1024, 8192
MC = M // N_DEV
NH = N // 2
OT = 1024


def _gemm(x, w):
    BM, BN = 1024, 2048

    def body(x_ref, w_ref, o_ref):
        a = x_ref[:, :].astype(jnp.bfloat16)
        b = w_ref[:, :].astype(jnp.bfloat16)
        o_ref[:, :] = jnp.dot(
            a, b, preferred_element_type=jnp.float32
        ).astype(jnp.bfloat16)

    return pl.pallas_call(
        body,
        grid=(M // BM, N // BN),
        in_specs=[
            pl.BlockSpec((BM, K), lambda i, j: (i, 0)),
            pl.BlockSpec((K, BN), lambda i, j: (0, j)),
        ],
        out_specs=pl.BlockSpec((BM, BN), lambda i, j: (i, j)),
        out_shape=jax.ShapeDtypeStruct((M, N), jnp.bfloat16),
    )(x, w)


def _ar_body(p_ref, scale_ref, out_ref, recv, pstage, ostage,
             send_sems, recv_sems, load_sems, ostage_sems, credit_sems):
    d = lax.axis_index("i")
    scale = scale_ref[0, 0]
    DIRS = (1, -1)
    halfbase = (0, NH)
    nbr = [jnp.mod(d + 1, N_DEV), jnp.mod(d - 1, N_DEV)]
    peer_out = [nbr[0], nbr[1]]
    peer_in = [nbr[1], nbr[0]]

    def rows(c):
        return pl.ds(c * MC, MC)

    preloads = []
    pfetch = [None, None]
    for k, dirn in enumerate(DIRS):
        cp = pltpu.make_async_copy(
            p_ref.at[rows(d), pl.ds(halfbase[k], NH)],
            recv.at[k, 0], load_sems.at[k, 0])
        cp.start()
        preloads.append(cp)
        cr0 = jnp.mod(d - dirn, N_DEV)
        cp = pltpu.make_async_copy(
            p_ref.at[rows(cr0), pl.ds(halfbase[k], NH)],
            pstage.at[k], load_sems.at[k, 1])
        cp.start()
        pfetch[k] = cp

    barrier = pltpu.get_barrier_semaphore()
    for k in range(2):
        pl.semaphore_signal(barrier, inc=1, device_id=(nbr[k],),
                            device_id_type=pl.DeviceIdType.MESH)
    pl.semaphore_wait(barrier, 2)
    for cp in preloads:
        cp.wait()

    last_ocp = [None, None]

    def epilogue(k, slot, c):
        for j in range(NH // OT):
            if last_ocp[k] is not None:
                last_ocp[k].wait()
            v = recv[k, slot, :, j * OT:(j + 1) * OT].astype(jnp.float32)
            ostage[k, :, :] = jnp.maximum(v * scale, 0.0)
            cp = pltpu.make_async_copy(
                ostage.at[k],
                out_ref.at[rows(c), pl.ds(halfbase[k] + j * OT, OT)],
                ostage_sems.at[k])
            cp.start()
            last_ocp[k] = cp

    rdmas = [[None] * 6 for _ in range(2)]
    for t in range(6):
        S, D = t % 2, (t + 1) % 2
        for k, dirn in enumerate(DIRS):
            if t >= 1:
                pl.semaphore_wait(credit_sems.at[k], 1)
            r = pltpu.make_async_remote_copy(
                src_ref=recv.at[k, S],
                dst_ref=recv.at[k, D],
                send_sem=send_sems.at[k, S],
                recv_sem=recv_sems.at[k, D],
                device_id=(peer_out[k],),
                device_id_type=pl.DeviceIdType.MESH)
            r.start()
            rdmas[k][t] = r
        if t - 1 >= 2:
            for k, dirn in enumerate(DIRS):
                tp = t - 1
                c = (jnp.mod(d + dirn, N_DEV) if tp == 2
                     else jnp.mod(d - dirn * (tp - 3), N_DEV))
                epilogue(k, (tp + 1) % 2, c)
        for k, dirn in enumerate(DIRS):
            if t <= 2:
                pfetch[k].wait()
            rdmas[k][t].wait_recv()
            if t <= 2:
                acc = (recv[k, D, :, :].astype(jnp.float32)
                       + pstage[k, :, :].astype(jnp.float32))
                recv[k, D, :, :] = acc.astype(jnp.bfloat16)
                if t < 2:
                    crn = jnp.mod(d - dirn * (t + 2), N_DEV)
                    cp = pltpu.make_async_copy(
                        p_ref.at[rows(crn), pl.ds(halfbase[k], NH)],
                        pstage.at[k], load_sems.at[k, 1])
                    cp.start()
                    pfetch[k] = cp
            rdmas[k][t].wait_send()
            pl.semaphore_signal(credit_sems.at[k], inc=1,
                                device_id=(peer_in[k],),
                                device_id_type=pl.DeviceIdType.MESH)
    for k, dirn in enumerate(DIRS):
        epilogue(k, 0, jnp.mod(d - dirn * 2, N_DEV))
    for k in range(2):
        if last_ocp[k] is not None:
            last_ocp[k].wait()
        pl.semaphore_wait(credit_sems.at[k], 1)


def _all_reduce_relu(partial, scale):
    return pl.pallas_call(
        _ar_body,
        in_specs=[
            pl.BlockSpec(memory_space=pl.ANY),
            pl.BlockSpec(memory_space=pltpu.SMEM),
        ],
        out_specs=pl.BlockSpec(memory_space=pl.ANY),
        out_shape=jax.ShapeDtypeStruct((M, N), jnp.float32),
        scratch_shapes=[
            pltpu.VMEM((2, 2, MC, NH), jnp.bfloat16),
            pltpu.VMEM((2, MC, NH), jnp.bfloat16),
            pltpu.VMEM((2, MC, OT), jnp.float32),
            pltpu.SemaphoreType.DMA((2, 2)),
            pltpu.SemaphoreType.DMA((2, 2)),
            pltpu.SemaphoreType.DMA((2, 2)),
            pltpu.SemaphoreType.DMA((2,)),
            pltpu.SemaphoreType.REGULAR((2,)),
        ],
        compiler_params=pltpu.CompilerParams(
            collective_id=0, vmem_limit_bytes=64 * 1024 * 1024),
    )(partial, scale)


def kernel(x, w_mat, scale_x, scale_w):
    partial = _gemm(x, w_mat)
    scale = (scale_x * scale_w).reshape(1, 1).astype(jnp.float32)
    return _all_reduce_relu(partial, scale)


# device time: 732424 ns/iter; 1.0450x vs baseline; 1.0450x over previous
import jax
import jax.numpy as jnp
from jax import lax
from jax.experimental import pallas as pl
from jax.experimental.pallas import tpu as pltpu

N_DEV = 4
M, K, N = 4096, 1024, 8192
MC = M // N_DEV
NH = N // 2
OT = 1024


def _gemm(x, w):
    BM, BN = 1024, 2048

    def body(x_ref, w_ref, o_ref):
        a = x_ref[:, :].astype(jnp.float8_e4m3fn)
        b = w_ref[:, :].astype(jnp.float8_e5m2)
        o_ref[:, :] = lax.dot_general(
            a, b, (((1,), (0,)), ((), ())),
            preferred_element_type=jnp.float32,
        ).astype(jnp.bfloat16)

    return pl.pallas_call(
        body,
        grid=(N // BN, M // BM),
        in_specs=[
            pl.BlockSpec((BM, K), lambda j, i: (i, 0)),
            pl.BlockSpec((K, BN), lambda j, i: (0, j)),
        ],
        out_specs=pl.BlockSpec((BM, BN), lambda j, i: (i, j)),
        out_shape=jax.ShapeDtypeStruct((M, N), jnp.bfloat16),
    )(x, w)


def _ar_body(p_ref, scale_ref, out_ref, recv, pstage, ostage,
             send_sems, recv_sems, load_sems, ostage_sems, credit_sems):
    d = lax.axis_index("i")
    scale = scale_ref[0, 0]
    DIRS = (1, -1)
    halfbase = (0, NH)
    nbr = [jnp.mod(d + 1, N_DEV), jnp.mod(d - 1, N_DEV)]
    peer_out = [nbr[0], nbr[1]]
    peer_in = [nbr[1], nbr[0]]

    def rows(c):
        return pl.ds(c * MC, MC)

    preloads = []
    pfetch = [None, None]
    for k, dirn in enumerate(DIRS):
        cp = pltpu.make_async_copy(
            p_ref.at[rows(d), pl.ds(halfbase[k], NH)],
            recv.at[k, 0], load_sems.at[k, 0])
        cp.start()
        preloads.append(cp)
        cr0 = jnp.mod(d - dirn, N_DEV)
        cp = pltpu.make_async_copy(
            p_ref.at[rows(cr0), pl.ds(halfbase[k], NH)],
            pstage.at[k], load_sems.at[k, 1])
        cp.start()
        pfetch[k] = cp

    barrier = pltpu.get_barrier_semaphore()
    for k in range(2):
        pl.semaphore_signal(barrier, inc=1, device_id=(nbr[k],),
                            device_id_type=pl.DeviceIdType.MESH)
    pl.semaphore_wait(barrier, 2)
    for cp in preloads:
        cp.wait()

    last_ocp = [None, None]

    def epilogue(k, slot, c):
        for j in range(NH // OT):
            if last_ocp[k] is not None:
                last_ocp[k].wait()
            v = recv[k, slot, :, j * OT:(j + 1) * OT].astype(jnp.float32)
            ostage[k, :, :] = jnp.maximum(v * scale, 0.0)
            cp = pltpu.make_async_copy(
                ostage.at[k],
                out_ref.at[rows(c), pl.ds(halfbase[k] + j * OT, OT)],
                ostage_sems.at[k])
            cp.start()
            last_ocp[k] = cp

    SUB = 2
    MS = MC // SUB
    rdmas = [[[None] * SUB for _ in range(6)] for _ in range(2)]
    for t in range(6):
        S, D = t % 2, (t + 1) % 2
        for b in range(SUB):
            for k, dirn in enumerate(DIRS):
                if t >= 1:
                    pl.semaphore_wait(credit_sems.at[k], 1)
                r = pltpu.make_async_remote_copy(
                    src_ref=recv.at[k, S, pl.ds(b * MS, MS)],
                    dst_ref=recv.at[k, D, pl.ds(b * MS, MS)],
                    send_sem=send_sems.at[k, S, b],
                    recv_sem=recv_sems.at[k, D, b],
                    device_id=(peer_out[k],),
                    device_id_type=pl.DeviceIdType.MESH)
                r.start()
                rdmas[k][t][b] = r
        if t - 1 >= 2:
            for k, dirn in enumerate(DIRS):
                tp = t - 1
                c = (jnp.mod(d + dirn, N_DEV) if tp == 2
                     else jnp.mod(d - dirn * (tp - 3), N_DEV))
                epilogue(k, (tp + 1) % 2, c)
        if t <= 2:
            for k in range(2):
                pfetch[k].wait()
        for b in range(SUB):
            for k, dirn in enumerate(DIRS):
                rdmas[k][t][b].wait_recv()
                if t <= 2:
                    rs = pl.ds(b * MS, MS)
                    acc = (recv[k, D, rs, :].astype(jnp.float32)
                           + pstage[k, rs, :].astype(jnp.float32))
                    recv[k, D, rs, :] = acc.astype(jnp.bfloat16)
                rdmas[k][t][b].wait_send()
                pl.semaphore_signal(credit_sems.at[k], inc=1,
                                    device_id=(peer_in[k],),
                                    device_id_type=pl.DeviceIdType.MESH)
        if t < 2:
            for k, dirn in enumerate(DIRS):
                crn = jnp.mod(d - dirn * (t + 2), N_DEV)
                cp = pltpu.make_async_copy(
                    p_ref.at[rows(crn), pl.ds(halfbase[k], NH)],
                    pstage.at[k], load_sems.at[k, 1])
                cp.start()
                pfetch[k] = cp
    for k, dirn in enumerate(DIRS):
        epilogue(k, 0, jnp.mod(d - dirn * 2, N_DEV))
    for k in range(2):
        if last_ocp[k] is not None:
            last_ocp[k].wait()
        pl.semaphore_wait(credit_sems.at[k], 2)


def _all_reduce_relu(partial, scale):
    return pl.pallas_call(
        _ar_body,
        in_specs=[
            pl.BlockSpec(memory_space=pl.ANY),
            pl.BlockSpec(memory_space=pltpu.SMEM),
        ],
        out_specs=pl.BlockSpec(memory_space=pl.ANY),
        out_shape=jax.ShapeDtypeStruct((M, N), jnp.float32),
        scratch_shapes=[
            pltpu.VMEM((2, 2, MC, NH), jnp.bfloat16),
            pltpu.VMEM((2, MC, NH), jnp.bfloat16),
            pltpu.VMEM((2, MC, OT), jnp.float32),
            pltpu.SemaphoreType.DMA((2, 2, 2)),
            pltpu.SemaphoreType.DMA((2, 2, 2)),
            pltpu.SemaphoreType.DMA((2, 2)),
            pltpu.SemaphoreType.DMA((2,)),
            pltpu.SemaphoreType.REGULAR((2,)),
        ],
        compiler_params=pltpu.CompilerParams(
            collective_id=0, vmem_limit_bytes=64 * 1024 * 1024),
    )(partial, scale)


def kernel(x, w_mat, scale_x, scale_w):
    partial = _gemm(x, w_mat)
    scale = (scale_x * scale_w).reshape(1, 1).astype(jnp.float32)
    return _all_reduce_relu(partial, scale)


# device time: 703070 ns/iter; 1.0887x vs baseline; 1.0418x over previous
import jax
import jax.numpy as jnp
from jax import lax
from jax.experimental import pallas as pl
from jax.experimental.pallas import tpu as pltpu

N_DEV = 4
M, K, N = 4096, 1024, 8192
MC = M // N_DEV
NH = N // 2
OT = 1024
SUB = 2
MS = MC // SUB
CJ = 1024
CT = 512


def _body(x_ref, w_ref, scale_ref, out_ref, recv, xq, wq, cstage, ostage,
          send_sems, recv_sems, conv_sems, ostage_sems, credit_sems):
    d = lax.axis_index("i")
    scale = scale_ref[0, 0]
    DIRS = (1, -1)
    halfbase = (0, NH)
    nbr = [jnp.mod(d + 1, N_DEV), jnp.mod(d - 1, N_DEV)]
    peer_out = [nbr[0], nbr[1]]
    peer_in = [nbr[1], nbr[0]]

    def rows(c):
        return pl.ds(c * MC, MC)

    n_xt = M // CT
    n_wt = (K // CT) * (N // 1024)
    tiles = []
    for i in range(n_xt):
        tiles.append(("x", pl.ds(i * CT, CT), slice(None)))
    for jc in range(N // 1024):
        for ir in range(K // CT):
            tiles.append(("w", pl.ds(ir * CT, CT), pl.ds(jc * 1024, 1024)))

    def conv_start(idx):
        kind, rsl, csl = tiles[idx]
        src = x_ref if kind == "x" else w_ref
        cp = pltpu.make_async_copy(
            src.at[rsl, csl], cstage.at[idx % 2], conv_sems.at[idx % 2])
        cp.start()
        return cp

    cps = {0: conv_start(0), 1: conv_start(1)}
    for idx in range(len(tiles)):
        kind, rsl, csl = tiles[idx]
        cps[idx].wait()
        if kind == "x":
            xq[rsl, :] = cstage[idx % 2].astype(jnp.float8_e4m3fn)
        else:
            wq[rsl, csl] = cstage[idx % 2].astype(jnp.float8_e5m2)
        if idx + 2 < len(tiles):
            cps[idx + 2] = conv_start(idx + 2)

    barrier = pltpu.get_barrier_semaphore()
    for k in range(2):
        pl.semaphore_signal(barrier, inc=1, device_id=(nbr[k],),
                            device_id_type=pl.DeviceIdType.MESH)
    pl.semaphore_wait(barrier, 2)

    def chunk_dot(xrows, k, j):
        return lax.dot_general(
            xq[xrows, :], wq[:, pl.ds(halfbase[k] + j * CJ, CJ)],
            (((1,), (0,)), ((), ())),
            preferred_element_type=jnp.float32)

    for k in range(2):
        for j in range(NH // CJ):
            recv[k, 0, :, j * CJ:(j + 1) * CJ] = chunk_dot(
                pl.ds(d * MC, MC), k, j).astype(jnp.bfloat16)

    last_ocp = [None, None]

    def epilogue(k, slot, c):
        for j in range(NH // OT):
            if last_ocp[k] is not None:
                last_ocp[k].wait()
            v = recv[k, slot, :, j * OT:(j + 1) * OT].astype(jnp.float32)
            ostage[k, :, :] = jnp.maximum(v * scale, 0.0)
            cp = pltpu.make_async_copy(
                ostage.at[k],
                out_ref.at[rows(c), pl.ds(halfbase[k] + j * OT, OT)],
                ostage_sems.at[k])
            cp.start()
            last_ocp[k] = cp

    rdmas = [[[None] * SUB for _ in range(6)] for _ in range(2)]
    for t in range(6):
        S, D = t % 2, (t + 1) % 2
        for b in range(SUB):
            for k, dirn in enumerate(DIRS):
                if t >= 1:
                    pl.semaphore_wait(credit_sems.at[k], 1)
                r = pltpu.make_async_remote_copy(
                    src_ref=recv.at[k, S, pl.ds(b * MS, MS)],
                    dst_ref=recv.at[k, D, pl.ds(b * MS, MS)],
                    send_sem=send_sems.at[k, S, b],
                    recv_sem=recv_sems.at[k, D, b],
                    device_id=(peer_out[k],),
                    device_id_type=pl.DeviceIdType.MESH)
                r.start()
                rdmas[k][t][b] = r
        if t - 1 >= 2:
            for k, dirn in enumerate(DIRS):
                tp = t - 1
                c = (jnp.mod(d + dirn, N_DEV) if tp == 2
                     else jnp.mod(d - dirn * (tp - 3), N_DEV))
                epilogue(k, (tp + 1) % 2, c)
        for b in range(SUB):
            for k, dirn in enumerate(DIRS):
                rdmas[k][t][b].wait_recv()
                if t <= 2:
                    crt = jnp.mod(d - dirn * (t + 1), N_DEV)
                    xrows = pl.ds(crt * MC + b * MS, MS)
                    rs = pl.ds(b * MS, MS)
                    for j in range(NH // CJ):
                        cs = slice(j * CJ, (j + 1) * CJ)
                        acc = (recv[k, D, rs, cs].astype(jnp.float32)
                               + chunk_dot(xrows, k, j))
                        recv[k, D, rs, cs] = acc.astype(jnp.bfloat16)
                rdmas[k][t][b].wait_send()
                pl.semaphore_signal(credit_sems.at[k], inc=1,
                                    device_id=(peer_in[k],),
                                    device_id_type=pl.DeviceIdType.MESH)
    for k, dirn in enumerate(DIRS):
        epilogue(k, 0, jnp.mod(d - dirn * 2, N_DEV))
    for k in range(2):
        if last_ocp[k] is not None:
            last_ocp[k].wait()
        pl.semaphore_wait(credit_sems.at[k], 2)


def kernel(x, w_mat, scale_x, scale_w):
    scale = (scale_x * scale_w).reshape(1, 1).astype(jnp.float32)
    return pl.pallas_call(
        _body,
        in_specs=[
            pl.BlockSpec(memory_space=pl.ANY),
            pl.BlockSpec(memory_space=pl.ANY),
            pl.BlockSpec(memory_space=pltpu.SMEM),
        ],
        out_specs=pl.BlockSpec(memory_space=pl.ANY),
        out_shape=jax.ShapeDtypeStruct((M, N), jnp.float32),
        scratch_shapes=[
            pltpu.VMEM((2, 2, MC, NH), jnp.bfloat16),
            pltpu.VMEM((M, K), jnp.float8_e4m3fn),
            pltpu.VMEM((K, N), jnp.float8_e5m2),
            pltpu.VMEM((2, CT, 1024), jnp.float32),
            pltpu.VMEM((2, MC, OT), jnp.float32),
            pltpu.SemaphoreType.DMA((2, 2, SUB)),
            pltpu.SemaphoreType.DMA((2, 2, SUB)),
            pltpu.SemaphoreType.DMA((2,)),
            pltpu.SemaphoreType.DMA((2,)),
            pltpu.SemaphoreType.REGULAR((2,)),
        ],
        compiler_params=pltpu.CompilerParams(
            collective_id=0, vmem_limit_bytes=64 * 1024 * 1024),
    )(x, w_mat, scale)


# device time: 673933 ns/iter; 1.1357x vs baseline; 1.0432x over previous
import jax
import jax.numpy as jnp
from jax import lax
from jax.experimental import pallas as pl
from jax.experimental.pallas import tpu as pltpu

N_DEV = 4
M, K, N = 4096, 1024, 8192
MC = M // N_DEV
NH = N // 2
OT = 1024
SUB = 2
MS = MC // SUB
CJ = 1024
CT = 512


def _body(x_ref, w_ref, scale_ref, out_ref, recv, xq, wq, cstage, ostage,
          send_sems, recv_sems, conv_sems, ostage_sems, credit_sems):
    d = lax.axis_index("i")
    scale = scale_ref[0, 0]
    DIRS = (1, -1)
    halfbase = (0, NH)
    nbr = [jnp.mod(d + 1, N_DEV), jnp.mod(d - 1, N_DEV)]
    peer_out = [nbr[0], nbr[1]]
    peer_in = [nbr[1], nbr[0]]

    def rows(c):
        return pl.ds(c * MC, MC)

    n_xt = M // CT
    n_wt = (K // CT) * (N // 1024)
    tiles = []
    for i in range(n_xt):
        tiles.append(("x", pl.ds(i * CT, CT), slice(None)))
    for jc in range(N // 1024):
        for ir in range(K // CT):
            tiles.append(("w", pl.ds(ir * CT, CT), pl.ds(jc * 1024, 1024)))

    def conv_start(idx):
        kind, rsl, csl = tiles[idx]
        src = x_ref if kind == "x" else w_ref
        cp = pltpu.make_async_copy(
            src.at[rsl, csl], cstage.at[idx % 2], conv_sems.at[idx % 2])
        cp.start()
        return cp

    cps = {0: conv_start(0), 1: conv_start(1)}
    for idx in range(len(tiles)):
        kind, rsl, csl = tiles[idx]
        cps[idx].wait()
        if kind == "x":
            xq[rsl, :] = cstage[idx % 2].astype(jnp.float8_e4m3fn)
        else:
            wq[rsl, csl] = cstage[idx % 2].astype(jnp.float8_e5m2)
        if idx + 2 < len(tiles):
            cps[idx + 2] = conv_start(idx + 2)

    barrier = pltpu.get_barrier_semaphore()
    for k in range(2):
        pl.semaphore_signal(barrier, inc=1, device_id=(nbr[k],),
                            device_id_type=pl.DeviceIdType.MESH)
    pl.semaphore_wait(barrier, 2)

    def chunk_dot(xrows, k, j):
        return lax.dot_general(
            xq[xrows, :], wq[:, pl.ds(halfbase[k] + j * CJ, CJ)],
            (((1,), (0,)), ((), ())),
            preferred_element_type=jnp.float32)

    def mk_rdma(k, t, b):
        S, D = t % 2, (t + 1) % 2
        return pltpu.make_async_remote_copy(
            src_ref=recv.at[k, S, pl.ds(b * MS, MS)],
            dst_ref=recv.at[k, D, pl.ds(b * MS, MS)],
            send_sem=send_sems.at[k, S, b],
            recv_sem=recv_sems.at[k, D, b],
            device_id=(peer_out[k],),
            device_id_type=pl.DeviceIdType.MESH)

    rdmas = [[[None] * SUB for _ in range(6)] for _ in range(2)]
    for b in range(SUB):
        for k in range(2):
            for j in range(NH // CJ):
                recv[k, 0, pl.ds(b * MS, MS), j * CJ:(j + 1) * CJ] = (
                    chunk_dot(pl.ds(d * MC + b * MS, MS), k, j)
                    .astype(jnp.bfloat16))
            r = mk_rdma(k, 0, b)
            r.start()
            rdmas[k][0][b] = r

    last_ocp = [None, None]

    def epilogue(k, slot, c):
        for j in range(NH // OT):
            if last_ocp[k] is not None:
                last_ocp[k].wait()
            v = recv[k, slot, :, j * OT:(j + 1) * OT].astype(jnp.float32)
            ostage[k, :, :] = jnp.maximum(v * scale, 0.0)
            cp = pltpu.make_async_copy(
                ostage.at[k],
                out_ref.at[rows(c), pl.ds(halfbase[k] + j * OT, OT)],
                ostage_sems.at[k])
            cp.start()
            last_ocp[k] = cp

    for t in range(6):
        D = (t + 1) % 2
        for b in range(SUB):
            for k, dirn in enumerate(DIRS):
                rdmas[k][t][b].wait_recv()
                if t <= 2:
                    crt = jnp.mod(d - dirn * (t + 1), N_DEV)
                    xrows = pl.ds(crt * MC + b * MS, MS)
                    rs = pl.ds(b * MS, MS)
                    for j in range(NH // CJ):
                        cs = slice(j * CJ, (j + 1) * CJ)
                        acc = (recv[k, D, rs, cs].astype(jnp.float32)
                               + chunk_dot(xrows, k, j))
                        recv[k, D, rs, cs] = acc.astype(jnp.bfloat16)
                rdmas[k][t][b].wait_send()
                pl.semaphore_signal(credit_sems.at[k], inc=1,
                                    device_id=(peer_in[k],),
                                    device_id_type=pl.DeviceIdType.MESH)
                if t < 5:
                    pl.semaphore_wait(credit_sems.at[k], 1)
                    r = mk_rdma(k, t + 1, b)
                    r.start()
                    rdmas[k][t + 1][b] = r
        if t >= 2:
            for k, dirn in enumerate(DIRS):
                c = (jnp.mod(d + dirn, N_DEV) if t == 2
                     else jnp.mod(d - dirn * (t - 3), N_DEV))
                epilogue(k, (t + 1) % 2, c)
    for k in range(2):
        if last_ocp[k] is not None:
            last_ocp[k].wait()
        pl.semaphore_wait(credit_sems.at[k], 2)


def kernel(x, w_mat, scale_x, scale_w):
    scale = (scale_x * scale_w).reshape(1, 1).astype(jnp.float32)
    return pl.pallas_call(
        _body,
        in_specs=[
            pl.BlockSpec(memory_space=pl.ANY),
            pl.BlockSpec(memory_space=pl.ANY),
            pl.BlockSpec(memory_space=pltpu.SMEM),
        ],
        out_specs=pl.BlockSpec(memory_space=pl.ANY),
        out_shape=jax.ShapeDtypeStruct((M, N), jnp.float32),
        scratch_shapes=[
            pltpu.VMEM((2, 2, MC, NH), jnp.bfloat16),
            pltpu.VMEM((M, K), jnp.float8_e4m3fn),
            pltpu.VMEM((K, N), jnp.float8_e5m2),
            pltpu.VMEM((2, CT, 1024), jnp.float32),
            pltpu.VMEM((2, MC, OT), jnp.float32),
            pltpu.SemaphoreType.DMA((2, 2, SUB)),
            pltpu.SemaphoreType.DMA((2, 2, SUB)),
            pltpu.SemaphoreType.DMA((2,)),
            pltpu.SemaphoreType.DMA((2,)),
            pltpu.SemaphoreType.REGULAR((2,)),
        ],
        compiler_params=pltpu.CompilerParams(
            collective_id=0, vmem_limit_bytes=64 * 1024 * 1024),
    )(x, w_mat, scale)


# device time: 672114 ns/iter; 1.1388x vs baseline; 1.0027x over previous
import jax
import jax.numpy as jnp
from jax import lax
from jax.experimental import pallas as pl
from jax.experimental.pallas import tpu as pltpu

N_DEV = 4
M, K, N = 4096, 1024, 8192
MC = M // N_DEV
NH = N // 2
OT = 1024
SUB = 2
MS = MC // SUB
CJ = 1024
CT = 512


def _body(x_ref, w_ref, scale_ref, out_ref, recv, xq, wq, cstage, ostage,
          send_sems, recv_sems, conv_sems, ostage_sems, credit_sems):
    d = lax.axis_index("i")
    scale = scale_ref[0, 0]
    DIRS = (1, -1)
    halfbase = (0, NH)
    nbr = [jnp.mod(d + 1, N_DEV), jnp.mod(d - 1, N_DEV)]
    peer_out = [nbr[0], nbr[1]]
    peer_in = [nbr[1], nbr[0]]

    def rows(c):
        return pl.ds(c * MC, MC)

    barrier = pltpu.get_barrier_semaphore()
    for k in range(2):
        pl.semaphore_signal(barrier, inc=1, device_id=(nbr[k],),
                            device_id_type=pl.DeviceIdType.MESH)
    pl.semaphore_wait(barrier, 2)

    def convert(tiles):
        cps = {}

        def start(i):
            src, dst, rsl, csl, dt = tiles[i]
            cp = pltpu.make_async_copy(
                src.at[rsl, csl], cstage.at[i % 2], conv_sems.at[i % 2])
            cp.start()
            cps[i] = cp

        for i in range(min(2, len(tiles))):
            start(i)
        for i in range(len(tiles)):
            src, dst, rsl, csl, dt = tiles[i]
            cps[i].wait()
            dst[rsl, csl] = cstage[i % 2].astype(dt)
            if i + 2 < len(tiles):
                start(i + 2)

    def x_tiles(chunk):
        return [(x_ref, xq, pl.ds(chunk * MC + i * CT, CT), slice(None),
                 jnp.float8_e4m3fn) for i in range(MC // CT)]

    def w_tiles(k):
        return [(w_ref, wq, pl.ds(ir * CT, CT),
                 pl.ds(halfbase[k] + jc * 1024, 1024), jnp.float8_e5m2)
                for jc in range(NH // 1024) for ir in range(K // CT)]

    def chunk_dot(xrows, k, j):
        return lax.dot_general(
            xq[xrows, :], wq[:, pl.ds(halfbase[k] + j * CJ, CJ)],
            (((1,), (0,)), ((), ())),
            preferred_element_type=jnp.float32)

    def mk_rdma(k, t, b):
        S, D = t % 2, (t + 1) % 2
        return pltpu.make_async_remote_copy(
            src_ref=recv.at[k, S, pl.ds(b * MS, MS)],
            dst_ref=recv.at[k, D, pl.ds(b * MS, MS)],
            send_sem=send_sems.at[k, S, b],
            recv_sem=recv_sems.at[k, D, b],
            device_id=(peer_out[k],),
            device_id_type=pl.DeviceIdType.MESH)

    rdmas = [[[None] * SUB for _ in range(6)] for _ in range(2)]
    convert(x_tiles(d))
    for k in range(2):
        convert(w_tiles(k))
        for b in range(SUB):
            for j in range(NH // CJ):
                recv[k, 0, pl.ds(b * MS, MS), j * CJ:(j + 1) * CJ] = (
                    chunk_dot(pl.ds(d * MC + b * MS, MS), k, j)
                    .astype(jnp.bfloat16))
            r = mk_rdma(k, 0, b)
            r.start()
            rdmas[k][0][b] = r
    convert(x_tiles(jnp.mod(d - 1, N_DEV))
            + x_tiles(jnp.mod(d + 1, N_DEV))
            + x_tiles(jnp.mod(d + 2, N_DEV)))

    last_ocp = [None, None]

    def epilogue(k, slot, c):
        for j in range(NH // OT):
            if last_ocp[k] is not None:
                last_ocp[k].wait()
            v = recv[k, slot, :, j * OT:(j + 1) * OT].astype(jnp.float32)
            ostage[k, :, :] = jnp.maximum(v * scale, 0.0)
            cp = pltpu.make_async_copy(
                ostage.at[k],
                out_ref.at[rows(c), pl.ds(halfbase[k] + j * OT, OT)],
                ostage_sems.at[k])
            cp.start()
            last_ocp[k] = cp

    for t in range(6):
        D = (t + 1) % 2
        for b in range(SUB):
            for k, dirn in enumerate(DIRS):
                rdmas[k][t][b].wait_recv()
                if t <= 2:
                    crt = jnp.mod(d - dirn * (t + 1), N_DEV)
                    xrows = pl.ds(crt * MC + b * MS, MS)
                    rs = pl.ds(b * MS, MS)
                    for j in range(NH // CJ):
                        cs = slice(j * CJ, (j + 1) * CJ)
                        acc = (recv[k, D, rs, cs].astype(jnp.float32)
                               + chunk_dot(xrows, k, j))
                        recv[k, D, rs, cs] = acc.astype(jnp.bfloat16)
                rdmas[k][t][b].wait_send()
                pl.semaphore_signal(credit_sems.at[k], inc=1,
                                    device_id=(peer_in[k],),
                                    device_id_type=pl.DeviceIdType.MESH)
                if t < 5:
                    pl.semaphore_wait(credit_sems.at[k], 1)
                    r = mk_rdma(k, t + 1, b)
                    r.start()
                    rdmas[k][t + 1][b] = r
        if t >= 2:
            for k, dirn in enumerate(DIRS):
                c = (jnp.mod(d + dirn, N_DEV) if t == 2
                     else jnp.mod(d - dirn * (t - 3), N_DEV))
                epilogue(k, (t + 1) % 2, c)
    for k in range(2):
        if last_ocp[k] is not None:
            last_ocp[k].wait()
        pl.semaphore_wait(credit_sems.at[k], 2)


def kernel(x, w_mat, scale_x, scale_w):
    scale = (scale_x * scale_w).reshape(1, 1).astype(jnp.float32)
    return pl.pallas_call(
        _body,
        in_specs=[
            pl.BlockSpec(memory_space=pl.ANY),
            pl.BlockSpec(memory_space=pl.ANY),
            pl.BlockSpec(memory_space=pltpu.SMEM),
        ],
        out_specs=pl.BlockSpec(memory_space=pl.ANY),
        out_shape=jax.ShapeDtypeStruct((M, N), jnp.float32),
        scratch_shapes=[
            pltpu.VMEM((2, 2, MC, NH), jnp.bfloat16),
            pltpu.VMEM((M, K), jnp.float8_e4m3fn),
            pltpu.VMEM((K, N), jnp.float8_e5m2),
            pltpu.VMEM((2, CT, 1024), jnp.float32),
            pltpu.VMEM((2, MC, OT), jnp.float32),
            pltpu.SemaphoreType.DMA((2, 2, SUB)),
            pltpu.SemaphoreType.DMA((2, 2, SUB)),
            pltpu.SemaphoreType.DMA((2,)),
            pltpu.SemaphoreType.DMA((2,)),
            pltpu.SemaphoreType.REGULAR((2,)),
        ],
        compiler_params=pltpu.CompilerParams(
            collective_id=0, vmem_limit_bytes=64 * 1024 * 1024),
    )(x, w_mat, scale)
